# baseline (speedup 1.0000x reference)
import jax
import jax.numpy as jnp
from jax import lax
from jax.experimental import pallas as pl
from jax.experimental.pallas import tpu as pltpu

_C = 96
_K = 8
_H = 224
_W = 224
_NCLS = 11
_NPAD = 16
_EPS = 1e-8
_NB = 4
_R = _H // _NB


def _body(v1_ref, v2_ref, mf_ref, oh_ref, ohT_ref, out_ref,
          accA, accG, acc_mq1, acc_cnt, acc_cr):
    t = pl.program_id(0)

    @pl.when(t == 0)
    def _init():
        accA[...] = jnp.zeros_like(accA)
        accG[...] = jnp.zeros_like(accG)
        acc_mq1[...] = jnp.zeros_like(acc_mq1)
        acc_cnt[...] = jnp.zeros_like(acc_cnt)
        acc_cr[...] = jnp.zeros_like(acc_cr)

    v1b = v1_ref[0].reshape(_C, _R * _W)
    v2b = v2_ref[0].reshape(_C, _R * _W)
    mf = mf_ref[0].astype(jnp.float32).reshape(_K, _R * _W)

    pn2 = jnp.sum(v2b * v2b, axis=0, keepdims=True)
    rinv = lax.rsqrt(jnp.maximum(pn2, _EPS * _EPS))
    q1 = jnp.sum(v1b * v1b, axis=0, keepdims=True)

    v1hi = v1b.astype(jnp.bfloat16).astype(jnp.float32)
    v1lo = v1b - v1hi
    w2 = v2b * rinv
    w2hi = w2.astype(jnp.bfloat16).astype(jnp.float32)
    w2lo = w2 - w2hi

    ct = (((1,), (1,)), ((), ()))
    accA[...] += (lax.dot_general(mf, v1hi, ct,
                                  preferred_element_type=jnp.float32) +
                  lax.dot_general(mf, v1lo, ct,
                                  preferred_element_type=jnp.float32))
    accG[...] += (lax.dot_general(mf, w2hi, ct,
                                  preferred_element_type=jnp.float32) +
                  lax.dot_general(mf, w2lo, ct,
                                  preferred_element_type=jnp.float32))
    acc_mq1[...] += lax.dot_general(mf, q1, ct,
                                    preferred_element_type=jnp.float32)
    ones_b = jnp.ones((1, _R * _W), jnp.float32)
    acc_cnt[...] += lax.dot_general(mf, ones_b, ct,
                                    preferred_element_type=jnp.float32)
    acc_cr[...] += lax.dot_general(ones_b, mf, ct,
                                   preferred_element_type=jnp.float32)

    @pl.when(t == _NB - 1)
    def _finalize():
        A = accA[...]
        G = accG[...]
        mq1 = acc_mq1[...]
        n = acc_cnt[...]
        nr = acc_cr[...]

        means = A / n
        mnorm = jnp.sqrt(jnp.sum(means * means, axis=1, keepdims=True))
        contract = (((1,), (1,)), ((), ()))
        Traw = lax.dot_general(means, G, contract,
                               preferred_element_type=jnp.float32,
                               precision=lax.Precision.HIGHEST)
        Ts = Traw / mnorm / nr

        eye = (lax.broadcasted_iota(jnp.int32, (_K, _K), 0) ==
               lax.broadcasted_iota(jnp.int32, (_K, _K), 1)).astype(jnp.float32)
        oh = oh_ref[...]
        ohT = ohT_ref[...]
        same = lax.dot_general(oh, ohT, (((1,), (0,)), ((), ())),
                               preferred_element_type=jnp.float32,
                               precision=lax.Precision.HIGHEST)

        binmm = (((1,), (0,)), ((), ()))
        diag_col = jnp.sum(Ts * eye, axis=1, keepdims=True)
        binsI = lax.dot_general(ohT, diag_col, binmm,
                                preferred_element_type=jnp.float32,
                                precision=lax.Precision.HIGHEST)
        off = same * (1.0 - eye)
        rowC = jnp.sum(Ts * off, axis=1, keepdims=True)
        binsC = lax.dot_general(ohT, rowC, binmm,
                                preferred_element_type=jnp.float32,
                                precision=lax.Precision.HIGHEST)
        negmask = 1.0 - same
        neg = jnp.sum(Ts * negmask) / jnp.sum(negmask)

        rowsA = jnp.sum(A, axis=1, keepdims=True)
        Cn = _C * n
        sq_dev = mq1 - rowsA * rowsA / Cn
        std_col = jnp.sqrt(sq_dev / (Cn - 1.0))
        binsS = lax.dot_general(ohT, std_col, binmm,
                                preferred_element_type=jnp.float32,
                                precision=lax.Precision.HIGHEST)

        cc = lax.dot_general(ohT, jnp.ones((_K, 1), jnp.float32), binmm,
                             preferred_element_type=jnp.float32,
                             precision=lax.Precision.HIGHEST)
        multi = cc > 1.0
        inst = jnp.where(multi, binsI / cc, binsI)
        clsm = jnp.where(multi, binsC / (cc * (cc - 1.0)), binsC)
        stdv = jnp.where(multi, binsS / cc, binsS)
        negcol = jnp.zeros((_NPAD, 1), jnp.float32) + neg
        pad = jnp.zeros((_NPAD, 4), jnp.float32)
        out_ref[...] = jnp.concatenate([inst, clsm, stdv, negcol, pad], axis=1)


def kernel(views_1, views_2, masks, labels):
    cls = labels[0]
    oh = (cls[:, None] == jnp.arange(_NPAD, dtype=cls.dtype)[None, :]
          ).astype(jnp.float32)
    ohT = oh.T

    res = pl.pallas_call(
        _body,
        grid=(_NB,),
        in_specs=[
            pl.BlockSpec((1, _C, _R, _W), lambda t: (0, 0, t, 0)),
            pl.BlockSpec((1, _C, _R, _W), lambda t: (0, 0, t, 0)),
            pl.BlockSpec((1, _K, _R, _W), lambda t: (0, 0, t, 0)),
            pl.BlockSpec((_K, _NPAD), lambda t: (0, 0)),
            pl.BlockSpec((_NPAD, _K), lambda t: (0, 0)),
        ],
        out_specs=pl.BlockSpec((_NPAD, _K), lambda t: (0, 0)),
        out_shape=jax.ShapeDtypeStruct((_NPAD, _K), jnp.float32),
        scratch_shapes=[
            pltpu.VMEM((_K, _C), jnp.float32),
            pltpu.VMEM((_K, _C), jnp.float32),
            pltpu.VMEM((_K, 1), jnp.float32),
            pltpu.VMEM((_K, 1), jnp.float32),
            pltpu.VMEM((1, _K), jnp.float32),
        ],
    )(views_1, views_2, masks, oh, ohT)

    instance_sim = res[:_NCLS, 0]
    class_sim = res[:_NCLS, 1]
    class_std = res[:_NCLS, 2]
    neg_sim = res[0:1, 3]
    return (instance_sim, class_sim, neg_sim, class_std)


# speedup vs baseline: 1.0118x; 1.0118x over previous
import jax
import jax.numpy as jnp
from jax import lax
from jax.experimental import pallas as pl
from jax.experimental.pallas import tpu as pltpu

_C = 96
_K = 8
_H = 224
_W = 224
_NCLS = 11
_NPAD = 16
_EPS = 1e-8
_NB = 7
_R = _H // _NB


def _body(v1_ref, v2_ref, mf_ref, oh_ref, ohT_ref, out_ref,
          accA, accG, acc_mq1, acc_cnt, acc_cr):
    t = pl.program_id(0)

    @pl.when(t == 0)
    def _init():
        accA[...] = jnp.zeros_like(accA)
        accG[...] = jnp.zeros_like(accG)
        acc_mq1[...] = jnp.zeros_like(acc_mq1)
        acc_cnt[...] = jnp.zeros_like(acc_cnt)
        acc_cr[...] = jnp.zeros_like(acc_cr)

    v1b = v1_ref[0].reshape(_C, _R * _W)
    v2b = v2_ref[0].reshape(_C, _R * _W)
    mf = mf_ref[0].astype(jnp.float32).reshape(_K, _R * _W)

    pn2 = jnp.sum(v2b * v2b, axis=0, keepdims=True)
    rinv = lax.rsqrt(jnp.maximum(pn2, _EPS * _EPS))
    q1 = jnp.sum(v1b * v1b, axis=0, keepdims=True)

    v1hi = v1b.astype(jnp.bfloat16).astype(jnp.float32)
    v1lo = v1b - v1hi
    w2 = v2b * rinv
    w2hi = w2.astype(jnp.bfloat16).astype(jnp.float32)
    w2lo = w2 - w2hi

    ct = (((1,), (1,)), ((), ()))
    accA[...] += (lax.dot_general(mf, v1hi, ct,
                                  preferred_element_type=jnp.float32) +
                  lax.dot_general(mf, v1lo, ct,
                                  preferred_element_type=jnp.float32))
    accG[...] += (lax.dot_general(mf, w2hi, ct,
                                  preferred_element_type=jnp.float32) +
                  lax.dot_general(mf, w2lo, ct,
                                  preferred_element_type=jnp.float32))
    acc_mq1[...] += lax.dot_general(mf, q1, ct,
                                    preferred_element_type=jnp.float32)
    ones_b = jnp.ones((1, _R * _W), jnp.float32)
    acc_cnt[...] += lax.dot_general(mf, ones_b, ct,
                                    preferred_element_type=jnp.float32)
    acc_cr[...] += lax.dot_general(ones_b, mf, ct,
                                   preferred_element_type=jnp.float32)

    @pl.when(t == _NB - 1)
    def _finalize():
        A = accA[...]
        G = accG[...]
        mq1 = acc_mq1[...]
        n = acc_cnt[...]
        nr = acc_cr[...]

        means = A / n
        mnorm = jnp.sqrt(jnp.sum(means * means, axis=1, keepdims=True))
        contract = (((1,), (1,)), ((), ()))
        Traw = lax.dot_general(means, G, contract,
                               preferred_element_type=jnp.float32,
                               precision=lax.Precision.HIGHEST)
        Ts = Traw / mnorm / nr

        eye = (lax.broadcasted_iota(jnp.int32, (_K, _K), 0) ==
               lax.broadcasted_iota(jnp.int32, (_K, _K), 1)).astype(jnp.float32)
        oh = oh_ref[...]
        ohT = ohT_ref[...]
        same = lax.dot_general(oh, ohT, (((1,), (0,)), ((), ())),
                               preferred_element_type=jnp.float32,
                               precision=lax.Precision.HIGHEST)

        binmm = (((1,), (0,)), ((), ()))
        diag_col = jnp.sum(Ts * eye, axis=1, keepdims=True)
        binsI = lax.dot_general(ohT, diag_col, binmm,
                                preferred_element_type=jnp.float32,
                                precision=lax.Precision.HIGHEST)
        off = same * (1.0 - eye)
        rowC = jnp.sum(Ts * off, axis=1, keepdims=True)
        binsC = lax.dot_general(ohT, rowC, binmm,
                                preferred_element_type=jnp.float32,
                                precision=lax.Precision.HIGHEST)
        negmask = 1.0 - same
        neg = jnp.sum(Ts * negmask) / jnp.sum(negmask)

        rowsA = jnp.sum(A, axis=1, keepdims=True)
        Cn = _C * n
        sq_dev = mq1 - rowsA * rowsA / Cn
        std_col = jnp.sqrt(sq_dev / (Cn - 1.0))
        binsS = lax.dot_general(ohT, std_col, binmm,
                                preferred_element_type=jnp.float32,
                                precision=lax.Precision.HIGHEST)

        cc = lax.dot_general(ohT, jnp.ones((_K, 1), jnp.float32), binmm,
                             preferred_element_type=jnp.float32,
                             precision=lax.Precision.HIGHEST)
        multi = cc > 1.0
        inst = jnp.where(multi, binsI / cc, binsI)
        clsm = jnp.where(multi, binsC / (cc * (cc - 1.0)), binsC)
        stdv = jnp.where(multi, binsS / cc, binsS)
        negcol = jnp.zeros((_NPAD, 1), jnp.float32) + neg
        pad = jnp.zeros((_NPAD, 4), jnp.float32)
        out_ref[...] = jnp.concatenate([inst, clsm, stdv, negcol, pad], axis=1)


def kernel(views_1, views_2, masks, labels):
    cls = labels[0]
    oh = (cls[:, None] == jnp.arange(_NPAD, dtype=cls.dtype)[None, :]
          ).astype(jnp.float32)
    ohT = oh.T

    res = pl.pallas_call(
        _body,
        grid=(_NB,),
        in_specs=[
            pl.BlockSpec((1, _C, _R, _W), lambda t: (0, 0, t, 0)),
            pl.BlockSpec((1, _C, _R, _W), lambda t: (0, 0, t, 0)),
            pl.BlockSpec((1, _K, _R, _W), lambda t: (0, 0, t, 0)),
            pl.BlockSpec((_K, _NPAD), lambda t: (0, 0)),
            pl.BlockSpec((_NPAD, _K), lambda t: (0, 0)),
        ],
        out_specs=pl.BlockSpec((_NPAD, _K), lambda t: (0, 0)),
        out_shape=jax.ShapeDtypeStruct((_NPAD, _K), jnp.float32),
        scratch_shapes=[
            pltpu.VMEM((_K, _C), jnp.float32),
            pltpu.VMEM((_K, _C), jnp.float32),
            pltpu.VMEM((_K, 1), jnp.float32),
            pltpu.VMEM((_K, 1), jnp.float32),
            pltpu.VMEM((1, _K), jnp.float32),
        ],
    )(views_1, views_2, masks, oh, ohT)

    instance_sim = res[:_NCLS, 0]
    class_sim = res[:_NCLS, 1]
    class_std = res[:_NCLS, 2]
    neg_sim = res[0:1, 3]
    return (instance_sim, class_sim, neg_sim, class_std)
